# baseline (speedup 1.0000x reference)
import jax
import jax.numpy as jnp
from jax.experimental import pallas as pl

L = 32768
H = 1024
K = 4
EPS = 1e-12
BLOCK = 2048


def _ln_body(idx_ref, in_ref, tab_ref, gam_ref, bet_ref, out_ref):
    idx = idx_ref[0, 0, :]
    x = in_ref[...]
    i2 = idx[:, None]
    pos = jnp.where(
        i2 < 2,
        jnp.where(i2 == 0, tab_ref[0:1, :], tab_ref[1:2, :]),
        jnp.where(i2 == 2, tab_ref[2:3, :], tab_ref[3:4, :]),
    )
    x = x + pos
    mu = jnp.mean(x, axis=-1, keepdims=True)
    var = jnp.mean(x * x, axis=-1, keepdims=True) - mu * mu
    y = (x - mu) * jax.lax.rsqrt(var + EPS)
    out_ref[...] = y * gam_ref[...] + bet_ref[...]


@jax.jit
def kernel(input_enc, emb_table, ln_gamma, ln_beta, embedding_index):
    nb = L // BLOCK
    idx3 = embedding_index.astype(jnp.int32).reshape(nb, 1, BLOCK)
    gam = ln_gamma.reshape(1, H)
    bet = ln_beta.reshape(1, H)
    return pl.pallas_call(
        _ln_body,
        grid=(nb,),
        in_specs=[
            pl.BlockSpec((1, 1, BLOCK), lambda i: (i, 0, 0)),
            pl.BlockSpec((BLOCK, H), lambda i: (i, 0)),
            pl.BlockSpec((K, H), lambda i: (0, 0)),
            pl.BlockSpec((1, H), lambda i: (0, 0)),
            pl.BlockSpec((1, H), lambda i: (0, 0)),
        ],
        out_specs=pl.BlockSpec((BLOCK, H), lambda i: (i, 0)),
        out_shape=jax.ShapeDtypeStruct((L, H), jnp.float32),
    )(idx3, input_enc, emb_table, gam, bet)


# speedup vs baseline: 1.0203x; 1.0203x over previous
import jax
import jax.numpy as jnp
from jax.experimental import pallas as pl

L = 32768
H = 1024
K = 4
EPS = 1e-12
BLOCK = 2048


def _ln_body(idx_ref, in_ref, tab_ref, gam_ref, bet_ref, out_ref):
    idx = idx_ref[0, 0, :]
    x = in_ref[...]
    tab = tab_ref[...]
    ks = jax.lax.broadcasted_iota(jnp.int32, (BLOCK, K), 1)
    onehot = (idx[:, None] == ks).astype(jnp.float32)
    pos = jnp.dot(onehot, tab, preferred_element_type=jnp.float32)
    x = x + pos
    mu = jnp.mean(x, axis=-1, keepdims=True)
    var = jnp.mean(x * x, axis=-1, keepdims=True) - mu * mu
    y = (x - mu) * jax.lax.rsqrt(var + EPS)
    out_ref[...] = y * gam_ref[...] + bet_ref[...]


@jax.jit
def kernel(input_enc, emb_table, ln_gamma, ln_beta, embedding_index):
    nb = L // BLOCK
    idx3 = embedding_index.astype(jnp.int32).reshape(nb, 1, BLOCK)
    gam = ln_gamma.reshape(1, H)
    bet = ln_beta.reshape(1, H)
    return pl.pallas_call(
        _ln_body,
        grid=(nb,),
        in_specs=[
            pl.BlockSpec((1, 1, BLOCK), lambda i: (i, 0, 0)),
            pl.BlockSpec((BLOCK, H), lambda i: (i, 0)),
            pl.BlockSpec((K, H), lambda i: (0, 0)),
            pl.BlockSpec((1, H), lambda i: (0, 0)),
            pl.BlockSpec((1, H), lambda i: (0, 0)),
        ],
        out_specs=pl.BlockSpec((BLOCK, H), lambda i: (i, 0)),
        out_shape=jax.ShapeDtypeStruct((L, H), jnp.float32),
    )(idx3, input_enc, emb_table, gam, bet)
